# Initial kernel scaffold; baseline (speedup 1.0000x reference)
import jax
import jax.numpy as jnp
from jax.experimental import pallas as pl


def kernel(x, edge_index, W1, al1, ar1, b1, p, W2, al2, ar2, b2, Wc, bc):
    raise NotImplementedError("write your pallas kernel here")


# speedup vs baseline: 18.7026x; 18.7026x over previous
import functools

import jax
import jax.numpy as jnp
from jax import lax
from jax.experimental import pallas as pl
from jax.experimental.pallas import tpu as pltpu
from jax.experimental.pallas import tpu_sc as plsc

N, E, D, C = 10000, 320000, 128, 40
AUG = 144
NC, NS = 2, 16
NW = NC * NS
EPW = E // NW
CH = 80
NCHUNK = EPW // CH
ROWS_PER_TILE = N // NS
ZR = 25
BN = 1000


def _proj_body(x_ref, w_ref, aler_ref, haug_ref, eler_ref):
    h = jnp.dot(x_ref[...], w_ref[...],
                preferred_element_type=jnp.float32,
                precision=lax.Precision.HIGHEST)
    haug_ref[:, :D] = h
    haug_ref[:, D:D + 1] = jnp.ones((BN, 1), jnp.float32)
    haug_ref[:, D + 1:] = jnp.zeros((BN, AUG - D - 1), jnp.float32)
    eler_ref[...] = jnp.dot(h, aler_ref[...],
                            preferred_element_type=jnp.float32,
                            precision=lax.Precision.HIGHEST)


def _proj1(x, w, aler):
    return pl.pallas_call(
        _proj_body,
        grid=(N // BN,),
        in_specs=[
            pl.BlockSpec((BN, D), lambda i: (i, 0)),
            pl.BlockSpec((D, D), lambda i: (0, 0)),
            pl.BlockSpec((D, 2), lambda i: (0, 0)),
        ],
        out_specs=[
            pl.BlockSpec((BN, AUG), lambda i: (i, 0)),
            pl.BlockSpec((BN, 2), lambda i: (i, 0)),
        ],
        out_shape=[
            jax.ShapeDtypeStruct((N, AUG), jnp.float32),
            jax.ShapeDtypeStruct((N, 2), jnp.float32),
        ],
    )(x, w, aler)


def _mid_body(pa_ref, pb_ref, b_ref, m_ref, w_ref, aler_ref, haug_ref, eler_ref):
    ps = pa_ref[...] + pb_ref[...]
    denom = ps[:, D:D + 1]
    out1 = ps[:, :D] / (denom + 1e-9) + b_ref[...]
    hin = jnp.where(out1 > 0, out1, jnp.exp(out1) - 1.0) * m_ref[...]
    h = jnp.dot(hin, w_ref[...],
                preferred_element_type=jnp.float32,
                precision=lax.Precision.HIGHEST)
    haug_ref[:, :D] = h
    haug_ref[:, D:D + 1] = jnp.ones((BN, 1), jnp.float32)
    haug_ref[:, D + 1:] = jnp.zeros((BN, AUG - D - 1), jnp.float32)
    eler_ref[...] = jnp.dot(h, aler_ref[...],
                            preferred_element_type=jnp.float32,
                            precision=lax.Precision.HIGHEST)


def _proj2(pa, pb, b, mask, w, aler):
    return pl.pallas_call(
        _mid_body,
        grid=(N // BN,),
        in_specs=[
            pl.BlockSpec((BN, AUG), lambda i: (i, 0)),
            pl.BlockSpec((BN, AUG), lambda i: (i, 0)),
            pl.BlockSpec((1, D), lambda i: (0, 0)),
            pl.BlockSpec((1, D), lambda i: (0, 0)),
            pl.BlockSpec((D, D), lambda i: (0, 0)),
            pl.BlockSpec((D, 2), lambda i: (0, 0)),
        ],
        out_specs=[
            pl.BlockSpec((BN, AUG), lambda i: (i, 0)),
            pl.BlockSpec((BN, 2), lambda i: (i, 0)),
        ],
        out_shape=[
            jax.ShapeDtypeStruct((N, AUG), jnp.float32),
            jax.ShapeDtypeStruct((N, 2), jnp.float32),
        ],
    )(pa, pb, b, mask, w, aler)


def _final_body(pa_ref, pb_ref, b_ref, wc_ref, bc_ref, out_ref, acc_ref):
    i = pl.program_id(0)
    ps = pa_ref[...] + pb_ref[...]
    denom = ps[:, D:D + 1]
    out2 = ps[:, :D] / (denom + 1e-9) + b_ref[...]
    s = jnp.sum(out2, axis=0, keepdims=True)

    @pl.when(i == 0)
    def _():
        acc_ref[...] = jnp.zeros_like(acc_ref)

    acc_ref[...] += s

    @pl.when(i == (N // BN) - 1)
    def _():
        out_ref[...] = jnp.dot(acc_ref[...] * (1.0 / N), wc_ref[...],
                               preferred_element_type=jnp.float32,
                               precision=lax.Precision.HIGHEST) + bc_ref[...]


def _final(pa, pb, b, wc, bc):
    return pl.pallas_call(
        _final_body,
        grid=(N // BN,),
        in_specs=[
            pl.BlockSpec((BN, AUG), lambda i: (i, 0)),
            pl.BlockSpec((BN, AUG), lambda i: (i, 0)),
            pl.BlockSpec((1, D), lambda i: (0, 0)),
            pl.BlockSpec((D, C), lambda i: (0, 0)),
            pl.BlockSpec((1, C), lambda i: (0, 0)),
        ],
        out_specs=pl.BlockSpec((1, C), lambda i: (0, 0)),
        out_shape=jax.ShapeDtypeStruct((1, C), jnp.float32),
        scratch_shapes=[pltpu.VMEM((1, D), jnp.float32)],
    )(pa, pb, b, wc, bc)


def _sc_body(haug_hbm, el_hbm, er_hbm, src_hbm, dst_hbm, part_hbm,
             out_shared, el_v, er_v, srcbuf, dstbuf, eebuf, rows, zbuf, sem):
    core = lax.axis_index("c")
    sub = lax.axis_index("s")
    wid = sub * NC + core

    z16 = jnp.zeros((16,), jnp.float32)

    @pl.loop(0, ZR)
    def _(rr):
        for j in range(AUG // 16):
            zbuf[rr, pl.ds(j * 16, 16)] = z16

    for k in range(ROWS_PER_TILE // ZR):
        row0 = sub * ROWS_PER_TILE + k * ZR
        pltpu.sync_copy(zbuf, out_shared.at[pl.ds(row0, ZR)])

    plsc.subcore_barrier()

    pltpu.sync_copy(el_hbm, el_v)
    pltpu.sync_copy(er_hbm, er_v)

    base = wid * EPW

    @pl.loop(0, NCHUNK)
    def _(c):
        off = base + c * CH
        pltpu.sync_copy(src_hbm.at[pl.ds(off, CH)], srcbuf)
        pltpu.sync_copy(dst_hbm.at[pl.ds(off, CH)], dstbuf)
        gcp = pltpu.async_copy(haug_hbm.at[srcbuf], rows, sem)

        for g in range(CH // 16):
            sv = srcbuf[pl.ds(g * 16, 16)]
            dv = dstbuf[pl.ds(g * 16, 16)]
            elv = plsc.load_gather(el_v, [sv])
            erv = plsc.load_gather(er_v, [dv])
            s = elv + erv
            e = jnp.where(s > 0.0, s, 0.2 * s)
            eebuf[pl.ds(g * 16, 16)] = jnp.exp(e)

        gcp.wait()

        @pl.loop(0, CH)
        def _(r):
            ridx = jnp.full((16,), 0, jnp.int32) + r
            ev = plsc.load_gather(eebuf, [ridx])
            for j in range(AUG // 16):
                rows[r, pl.ds(j * 16, 16)] = rows[r, pl.ds(j * 16, 16)] * ev

        pltpu.sync_copy(rows, out_shared.at[dstbuf], add=True)

    plsc.subcore_barrier()

    for k in range(ROWS_PER_TILE // ZR):
        row0 = sub * ROWS_PER_TILE + k * ZR
        pltpu.sync_copy(out_shared.at[pl.ds(row0, ZR)],
                        part_hbm.at[core, pl.ds(row0, ZR)])


def _sc_aggregate(haug, el, er, src, dst):
    mesh = plsc.VectorSubcoreMesh(core_axis_name="c", subcore_axis_name="s")
    f = functools.partial(
        pl.kernel,
        mesh=mesh,
        compiler_params=pltpu.CompilerParams(use_tc_tiling_on_sc=False,
                                             needs_layout_passes=False),
        out_type=jax.ShapeDtypeStruct((NC, N, AUG), jnp.float32),
        scratch_types=[
            pltpu.VMEM_SHARED((N, AUG), jnp.float32),
            pltpu.VMEM((N,), jnp.float32),
            pltpu.VMEM((N,), jnp.float32),
            pltpu.VMEM((CH,), jnp.int32),
            pltpu.VMEM((CH,), jnp.int32),
            pltpu.VMEM((CH,), jnp.float32),
            pltpu.VMEM((CH, AUG), jnp.float32),
            pltpu.VMEM((ZR, AUG), jnp.float32),
            pltpu.SemaphoreType.DMA,
        ],
    )(_sc_body)
    return f(haug, el, er, src, dst)


@jax.jit
def kernel(x, edge_index, W1, al1, ar1, b1, p, W2, al2, ar2, b2, Wc, bc):
    src = edge_index[0].astype(jnp.int32)
    dst = edge_index[1].astype(jnp.int32)
    aler1 = jnp.stack([al1, ar1], axis=1)
    aler2 = jnp.stack([al2, ar2], axis=1)
    mask = jnp.clip(p, 0.0, 1.0).reshape(1, D)

    haug1, eler1 = _proj1(x, W1, aler1)
    part1 = _sc_aggregate(haug1, eler1[:, 0], eler1[:, 1], src, dst)
    haug2, eler2 = _proj2(part1[0], part1[1], b1.reshape(1, D), mask, W2, aler2)
    part2 = _sc_aggregate(haug2, eler2[:, 0], eler2[:, 1], src, dst)
    out = _final(part2[0], part2[1], b2.reshape(1, D), Wc, bc.reshape(1, C))
    return out.reshape(C)


# speedup vs baseline: 38.2660x; 2.0460x over previous
import functools

import jax
import jax.numpy as jnp
from jax import lax
from jax.experimental import pallas as pl
from jax.experimental.pallas import tpu as pltpu
from jax.experimental.pallas import tpu_sc as plsc

N, E, D, C = 10000, 320000, 128, 40
NC, NS = 2, 16
NW = NC * NS
EPW = E // NW
CH = 80
NCHUNK = EPW // CH
CPB = 25
NBLK = NCHUNK // CPB
PAIRS = (CPB - 1) // 2
RPT = N // NS
DZ = 624
BN = 1000


def _proj_body(x_ref, w_ref, aler_ref, h_ref, eler_ref):
    h = jnp.dot(x_ref[...], w_ref[...],
                preferred_element_type=jnp.float32,
                precision=lax.Precision.HIGHEST)
    h_ref[...] = h
    eler_ref[...] = jnp.dot(h, aler_ref[...],
                            preferred_element_type=jnp.float32,
                            precision=lax.Precision.HIGHEST)


def _proj1(x, w, aler):
    return pl.pallas_call(
        _proj_body,
        grid=(N // BN,),
        in_specs=[
            pl.BlockSpec((BN, D), lambda i: (i, 0)),
            pl.BlockSpec((D, D), lambda i: (0, 0)),
            pl.BlockSpec((D, 2), lambda i: (0, 0)),
        ],
        out_specs=[
            pl.BlockSpec((BN, D), lambda i: (i, 0)),
            pl.BlockSpec((BN, 2), lambda i: (i, 0)),
        ],
        out_shape=[
            jax.ShapeDtypeStruct((N, D), jnp.float32),
            jax.ShapeDtypeStruct((N, 2), jnp.float32),
        ],
    )(x, w, aler)


def _mid_body(pa_ref, pb_ref, dna_ref, dnb_ref, b_ref, m_ref, w_ref, aler_ref,
              h_ref, eler_ref):
    ps = pa_ref[...] + pb_ref[...]
    dn = dna_ref[...] + dnb_ref[...]
    out1 = ps / (dn + 1e-9) + b_ref[...]
    hin = jnp.where(out1 > 0, out1, jnp.exp(out1) - 1.0) * m_ref[...]
    h = jnp.dot(hin, w_ref[...],
                preferred_element_type=jnp.float32,
                precision=lax.Precision.HIGHEST)
    h_ref[...] = h
    eler_ref[...] = jnp.dot(h, aler_ref[...],
                            preferred_element_type=jnp.float32,
                            precision=lax.Precision.HIGHEST)


def _proj2(pa, pb, dna, dnb, b, mask, w, aler):
    return pl.pallas_call(
        _mid_body,
        grid=(N // BN,),
        in_specs=[
            pl.BlockSpec((BN, D), lambda i: (i, 0)),
            pl.BlockSpec((BN, D), lambda i: (i, 0)),
            pl.BlockSpec((BN, 1), lambda i: (i, 0)),
            pl.BlockSpec((BN, 1), lambda i: (i, 0)),
            pl.BlockSpec((1, D), lambda i: (0, 0)),
            pl.BlockSpec((1, D), lambda i: (0, 0)),
            pl.BlockSpec((D, D), lambda i: (0, 0)),
            pl.BlockSpec((D, 2), lambda i: (0, 0)),
        ],
        out_specs=[
            pl.BlockSpec((BN, D), lambda i: (i, 0)),
            pl.BlockSpec((BN, 2), lambda i: (i, 0)),
        ],
        out_shape=[
            jax.ShapeDtypeStruct((N, D), jnp.float32),
            jax.ShapeDtypeStruct((N, 2), jnp.float32),
        ],
    )(pa, pb, dna, dnb, b, mask, w, aler)


def _final_body(pa_ref, pb_ref, dna_ref, dnb_ref, b_ref, wc_ref, bc_ref,
                out_ref, acc_ref):
    i = pl.program_id(0)
    ps = pa_ref[...] + pb_ref[...]
    dn = dna_ref[...] + dnb_ref[...]
    out2 = ps / (dn + 1e-9) + b_ref[...]
    s = jnp.sum(out2, axis=0, keepdims=True)

    @pl.when(i == 0)
    def _():
        acc_ref[...] = jnp.zeros_like(acc_ref)

    acc_ref[...] += s

    @pl.when(i == (N // BN) - 1)
    def _():
        out_ref[...] = jnp.dot(acc_ref[...] * (1.0 / N), wc_ref[...],
                               preferred_element_type=jnp.float32,
                               precision=lax.Precision.HIGHEST) + bc_ref[...]


def _final(pa, pb, dna, dnb, b, wc, bc):
    return pl.pallas_call(
        _final_body,
        grid=(N // BN,),
        in_specs=[
            pl.BlockSpec((BN, D), lambda i: (i, 0)),
            pl.BlockSpec((BN, D), lambda i: (i, 0)),
            pl.BlockSpec((BN, 1), lambda i: (i, 0)),
            pl.BlockSpec((BN, 1), lambda i: (i, 0)),
            pl.BlockSpec((1, D), lambda i: (0, 0)),
            pl.BlockSpec((D, C), lambda i: (0, 0)),
            pl.BlockSpec((1, C), lambda i: (0, 0)),
        ],
        out_specs=pl.BlockSpec((1, C), lambda i: (0, 0)),
        out_shape=jax.ShapeDtypeStruct((1, C), jnp.float32),
        scratch_shapes=[pltpu.VMEM((1, D), jnp.float32)],
    )(pa, pb, dna, dnb, b, wc, bc)


def _sc_body(h_hbm, el_hbm, er_hbm, src_hbm, dst_hbm, part_hbm, pdn_hbm,
             out_sh, dn_sh, el_v, er_v, srcb, dstb, eeA, eeB, rowsA, rowsB,
             gsA, gsB, ssA, ssB):
    core = lax.axis_index("c")
    sub = lax.axis_index("s")
    wid = sub * NC + core

    z16 = jnp.zeros((16,), jnp.float32)

    @pl.loop(0, CH)
    def _(r):
        for j in range(D // 16):
            rowsA[r, pl.ds(j * 16, 16)] = z16

    row0 = sub * RPT
    for k in range(RPT // CH):
        pltpu.sync_copy(rowsA, out_sh.at[pl.ds(row0 + k * CH, CH)])
    pltpu.sync_copy(rowsA.at[pl.ds(0, RPT % CH)],
                    out_sh.at[pl.ds(row0 + (RPT // CH) * CH, RPT % CH)])

    for g in range(CH // 16):
        eeA[pl.ds(g * 16, 16)] = z16
    d0 = sub * DZ
    for k in range(DZ // CH):
        pltpu.sync_copy(eeA, dn_sh.at[pl.ds(d0 + k * CH, CH)])
    pltpu.sync_copy(eeA.at[pl.ds(0, DZ % CH)],
                    dn_sh.at[pl.ds(d0 + (DZ // CH) * CH, DZ % CH)])

    @pl.when(sub == NS - 1)
    def _():
        pltpu.sync_copy(eeA.at[pl.ds(0, N - NS * DZ)],
                        dn_sh.at[pl.ds(NS * DZ, N - NS * DZ)])

    plsc.subcore_barrier()

    pltpu.sync_copy(el_hbm, el_v)
    pltpu.sync_copy(er_hbm, er_v)

    def compute_ee(jrow, eebuf):
        for g in range(CH // 16):
            sv = srcb[jrow, pl.ds(g * 16, 16)]
            dv = dstb[jrow, pl.ds(g * 16, 16)]
            s = plsc.load_gather(el_v, [sv]) + plsc.load_gather(er_v, [dv])
            e = jnp.where(s > 0.0, s, 0.2 * s)
            eebuf[pl.ds(g * 16, 16)] = jnp.exp(e)

    def scale_rows(rows, eebuf):
        @pl.loop(0, CH)
        def _(r):
            ridx = jnp.full((16,), 0, jnp.int32) + r
            ev = plsc.load_gather(eebuf, [ridx])
            for j in range(D // 16):
                rows[r, pl.ds(j * 16, 16)] = rows[r, pl.ds(j * 16, 16)] * ev

    @pl.loop(0, NBLK)
    def _(b):
        blk = wid * NCHUNK + b * CPB
        pltpu.sync_copy(src_hbm.at[pl.ds(blk, CPB)], srcb)
        pltpu.sync_copy(dst_hbm.at[pl.ds(blk, CPB)], dstb)

        pltpu.async_copy(h_hbm.at[srcb.at[0]], rowsA, gsA)

        @pl.loop(0, PAIRS)
        def _(i):
            j0 = 2 * i
            j1 = j0 + 1
            compute_ee(j0, eeA)
            pltpu.make_async_copy(h_hbm.at[srcb.at[j0]], rowsA, gsA).wait()

            @pl.when(i > 0)
            def _():
                pltpu.make_async_copy(rowsB, out_sh.at[dstb.at[j1]], ssB).wait()

            pltpu.async_copy(h_hbm.at[srcb.at[j1]], rowsB, gsB)
            scale_rows(rowsA, eeA)
            pltpu.async_copy(rowsA, out_sh.at[dstb.at[j0]], ssA, add=True)
            pltpu.sync_copy(eeA, dn_sh.at[dstb.at[j0]], add=True)

            compute_ee(j1, eeB)
            pltpu.make_async_copy(h_hbm.at[srcb.at[j1]], rowsB, gsB).wait()
            pltpu.make_async_copy(rowsA, out_sh.at[dstb.at[j0]], ssA).wait()
            pltpu.async_copy(h_hbm.at[srcb.at[j0 + 2]], rowsA, gsA)
            scale_rows(rowsB, eeB)
            pltpu.async_copy(rowsB, out_sh.at[dstb.at[j1]], ssB, add=True)
            pltpu.sync_copy(eeB, dn_sh.at[dstb.at[j1]], add=True)

        jt = CPB - 1
        compute_ee(jt, eeA)
        pltpu.make_async_copy(h_hbm.at[srcb.at[jt]], rowsA, gsA).wait()
        pltpu.make_async_copy(rowsB, out_sh.at[dstb.at[jt]], ssB).wait()
        scale_rows(rowsA, eeA)
        pltpu.sync_copy(rowsA, out_sh.at[dstb.at[jt]], add=True)
        pltpu.sync_copy(eeA, dn_sh.at[dstb.at[jt]], add=True)

    plsc.subcore_barrier()

    pltpu.sync_copy(out_sh.at[pl.ds(row0, RPT)],
                    part_hbm.at[core, pl.ds(row0, RPT)])
    pltpu.sync_copy(dn_sh.at[pl.ds(d0, DZ)],
                    pdn_hbm.at[core, pl.ds(d0, DZ)])

    @pl.when(sub == NS - 1)
    def _():
        pltpu.sync_copy(dn_sh.at[pl.ds(NS * DZ, N - NS * DZ)],
                        pdn_hbm.at[core, pl.ds(NS * DZ, N - NS * DZ)])


def _sc_aggregate(h, el, er, src2d, dst2d):
    mesh = plsc.VectorSubcoreMesh(core_axis_name="c", subcore_axis_name="s")
    f = functools.partial(
        pl.kernel,
        mesh=mesh,
        compiler_params=pltpu.CompilerParams(use_tc_tiling_on_sc=False,
                                             needs_layout_passes=False),
        out_type=[
            jax.ShapeDtypeStruct((NC, N, D), jnp.float32),
            jax.ShapeDtypeStruct((NC, N), jnp.float32),
        ],
        scratch_types=[
            pltpu.VMEM_SHARED((N, D), jnp.float32),
            pltpu.VMEM_SHARED((N,), jnp.float32),
            pltpu.VMEM((N,), jnp.float32),
            pltpu.VMEM((N,), jnp.float32),
            pltpu.VMEM((CPB, CH), jnp.int32),
            pltpu.VMEM((CPB, CH), jnp.int32),
            pltpu.VMEM((CH,), jnp.float32),
            pltpu.VMEM((CH,), jnp.float32),
            pltpu.VMEM((CH, D), jnp.float32),
            pltpu.VMEM((CH, D), jnp.float32),
            pltpu.SemaphoreType.DMA,
            pltpu.SemaphoreType.DMA,
            pltpu.SemaphoreType.DMA,
            pltpu.SemaphoreType.DMA,
        ],
    )(_sc_body)
    return f(h, el, er, src2d, dst2d)


@jax.jit
def kernel(x, edge_index, W1, al1, ar1, b1, p, W2, al2, ar2, b2, Wc, bc):
    src2d = edge_index[0].astype(jnp.int32).reshape(E // CH, CH)
    dst2d = edge_index[1].astype(jnp.int32).reshape(E // CH, CH)
    aler1 = jnp.stack([al1, ar1], axis=1)
    aler2 = jnp.stack([al2, ar2], axis=1)
    mask = jnp.clip(p, 0.0, 1.0).reshape(1, D)

    h1, eler1 = _proj1(x, W1, aler1)
    part1, pdn1 = _sc_aggregate(h1, eler1[:, 0], eler1[:, 1], src2d, dst2d)
    h2, eler2 = _proj2(part1[0], part1[1],
                       pdn1[0].reshape(N, 1), pdn1[1].reshape(N, 1),
                       b1.reshape(1, D), mask, W2, aler2)
    part2, pdn2 = _sc_aggregate(h2, eler2[:, 0], eler2[:, 1], src2d, dst2d)
    out = _final(part2[0], part2[1],
                 pdn2[0].reshape(N, 1), pdn2[1].reshape(N, 1),
                 b2.reshape(1, D), Wc, bc.reshape(1, C))
    return out.reshape(C)


# speedup vs baseline: 38.3035x; 1.0010x over previous
import functools

import jax
import jax.numpy as jnp
from jax import lax
from jax.experimental import pallas as pl
from jax.experimental.pallas import tpu as pltpu
from jax.experimental.pallas import tpu_sc as plsc

N, E, D, C = 10000, 320000, 128, 40
NC, NS = 2, 16
NW = NC * NS
EPW = E // NW
CH = 80
NCHUNK = EPW // CH
CPB = 25
NBLK = NCHUNK // CPB
PAIRS = (CPB - 1) // 2
RPT = N // NS
DZ = 624
BN = 1000


def _proj_body(x_ref, w_ref, aler_ref, h_ref, eler_ref):
    h = jnp.dot(x_ref[...], w_ref[...],
                preferred_element_type=jnp.float32,
                precision=lax.Precision.HIGHEST)
    h_ref[...] = h
    eler_ref[...] = jnp.dot(h, aler_ref[...],
                            preferred_element_type=jnp.float32,
                            precision=lax.Precision.HIGHEST)


def _proj1(x, w, aler):
    return pl.pallas_call(
        _proj_body,
        grid=(N // BN,),
        in_specs=[
            pl.BlockSpec((BN, D), lambda i: (i, 0)),
            pl.BlockSpec((D, D), lambda i: (0, 0)),
            pl.BlockSpec((D, 2), lambda i: (0, 0)),
        ],
        out_specs=[
            pl.BlockSpec((BN, D), lambda i: (i, 0)),
            pl.BlockSpec((BN, 2), lambda i: (i, 0)),
        ],
        out_shape=[
            jax.ShapeDtypeStruct((N, D), jnp.float32),
            jax.ShapeDtypeStruct((N, 2), jnp.float32),
        ],
    )(x, w, aler)


def _mid_body(pa_ref, pb_ref, dna_ref, dnb_ref, b_ref, m_ref, w_ref, aler_ref,
              h_ref, eler_ref):
    ps = pa_ref[...] + pb_ref[...]
    dn = dna_ref[...] + dnb_ref[...]
    out1 = ps / (dn + 1e-9) + b_ref[...]
    hin = jnp.where(out1 > 0, out1, jnp.exp(out1) - 1.0) * m_ref[...]
    h = jnp.dot(hin, w_ref[...],
                preferred_element_type=jnp.float32,
                precision=lax.Precision.HIGHEST)
    h_ref[...] = h
    eler_ref[...] = jnp.dot(h, aler_ref[...],
                            preferred_element_type=jnp.float32,
                            precision=lax.Precision.HIGHEST)


def _proj2(pa, pb, dna, dnb, b, mask, w, aler):
    return pl.pallas_call(
        _mid_body,
        grid=(N // BN,),
        in_specs=[
            pl.BlockSpec((BN, D), lambda i: (i, 0)),
            pl.BlockSpec((BN, D), lambda i: (i, 0)),
            pl.BlockSpec((BN, 1), lambda i: (i, 0)),
            pl.BlockSpec((BN, 1), lambda i: (i, 0)),
            pl.BlockSpec((1, D), lambda i: (0, 0)),
            pl.BlockSpec((1, D), lambda i: (0, 0)),
            pl.BlockSpec((D, D), lambda i: (0, 0)),
            pl.BlockSpec((D, 2), lambda i: (0, 0)),
        ],
        out_specs=[
            pl.BlockSpec((BN, D), lambda i: (i, 0)),
            pl.BlockSpec((BN, 2), lambda i: (i, 0)),
        ],
        out_shape=[
            jax.ShapeDtypeStruct((N, D), jnp.float32),
            jax.ShapeDtypeStruct((N, 2), jnp.float32),
        ],
    )(pa, pb, dna, dnb, b, mask, w, aler)


def _final_body(pa_ref, pb_ref, dna_ref, dnb_ref, b_ref, wc_ref, bc_ref,
                out_ref, acc_ref):
    i = pl.program_id(0)
    ps = pa_ref[...] + pb_ref[...]
    dn = dna_ref[...] + dnb_ref[...]
    out2 = ps / (dn + 1e-9) + b_ref[...]
    s = jnp.sum(out2, axis=0, keepdims=True)

    @pl.when(i == 0)
    def _():
        acc_ref[...] = jnp.zeros_like(acc_ref)

    acc_ref[...] += s

    @pl.when(i == (N // BN) - 1)
    def _():
        out_ref[...] = jnp.dot(acc_ref[...] * (1.0 / N), wc_ref[...],
                               preferred_element_type=jnp.float32,
                               precision=lax.Precision.HIGHEST) + bc_ref[...]


def _final(pa, pb, dna, dnb, b, wc, bc):
    return pl.pallas_call(
        _final_body,
        grid=(N // BN,),
        in_specs=[
            pl.BlockSpec((BN, D), lambda i: (i, 0)),
            pl.BlockSpec((BN, D), lambda i: (i, 0)),
            pl.BlockSpec((BN, 1), lambda i: (i, 0)),
            pl.BlockSpec((BN, 1), lambda i: (i, 0)),
            pl.BlockSpec((1, D), lambda i: (0, 0)),
            pl.BlockSpec((D, C), lambda i: (0, 0)),
            pl.BlockSpec((1, C), lambda i: (0, 0)),
        ],
        out_specs=pl.BlockSpec((1, C), lambda i: (0, 0)),
        out_shape=jax.ShapeDtypeStruct((1, C), jnp.float32),
        scratch_shapes=[pltpu.VMEM((1, D), jnp.float32)],
    )(pa, pb, dna, dnb, b, wc, bc)


def _sc_body(h_hbm, el_hbm, er_hbm, src_hbm, dst_hbm, part_hbm, pdn_hbm,
             out_sh, dn_sh, el_v, er_v, srcb, dstb, eeA, eeB, rowsA, rowsB,
             gsA, gsB, ssA, ssB, dsA, dsB):
    core = lax.axis_index("c")
    sub = lax.axis_index("s")
    wid = sub * NC + core

    z16 = jnp.zeros((16,), jnp.float32)

    @pl.loop(0, CH)
    def _(r):
        for j in range(D // 16):
            rowsA[r, pl.ds(j * 16, 16)] = z16

    row0 = sub * RPT
    for k in range(RPT // CH):
        pltpu.sync_copy(rowsA, out_sh.at[pl.ds(row0 + k * CH, CH)])
    pltpu.sync_copy(rowsA.at[pl.ds(0, RPT % CH)],
                    out_sh.at[pl.ds(row0 + (RPT // CH) * CH, RPT % CH)])

    for g in range(CH // 16):
        eeA[pl.ds(g * 16, 16)] = z16
    d0 = sub * DZ
    for k in range(DZ // CH):
        pltpu.sync_copy(eeA, dn_sh.at[pl.ds(d0 + k * CH, CH)])
    pltpu.sync_copy(eeA.at[pl.ds(0, DZ % CH)],
                    dn_sh.at[pl.ds(d0 + (DZ // CH) * CH, DZ % CH)])

    @pl.when(sub == NS - 1)
    def _():
        pltpu.sync_copy(eeA.at[pl.ds(0, N - NS * DZ)],
                        dn_sh.at[pl.ds(NS * DZ, N - NS * DZ)])

    plsc.subcore_barrier()

    pltpu.sync_copy(el_hbm, el_v)
    pltpu.sync_copy(er_hbm, er_v)

    def compute_ee(jrow, eebuf):
        for g in range(CH // 16):
            sv = srcb[jrow, pl.ds(g * 16, 16)]
            dv = dstb[jrow, pl.ds(g * 16, 16)]
            s = plsc.load_gather(el_v, [sv]) + plsc.load_gather(er_v, [dv])
            e = jnp.where(s > 0.0, s, 0.2 * s)
            eebuf[pl.ds(g * 16, 16)] = jnp.exp(e)

    def scale_rows(rows, eebuf):
        @pl.loop(0, CH)
        def _(r):
            ridx = jnp.full((16,), 0, jnp.int32) + r
            ev = plsc.load_gather(eebuf, [ridx])
            for j in range(D // 16):
                rows[r, pl.ds(j * 16, 16)] = rows[r, pl.ds(j * 16, 16)] * ev

    @pl.loop(0, NBLK)
    def _(b):
        blk = wid * NCHUNK + b * CPB

        @pl.when(b > 0)
        def _():
            pltpu.make_async_copy(eeB, dn_sh.at[dstb.at[0]], dsB).wait()

        pltpu.sync_copy(src_hbm.at[pl.ds(blk, CPB)], srcb)
        pltpu.sync_copy(dst_hbm.at[pl.ds(blk, CPB)], dstb)

        pltpu.async_copy(h_hbm.at[srcb.at[0]], rowsA, gsA)

        @pl.loop(0, PAIRS)
        def _(i):
            j0 = 2 * i
            j1 = j0 + 1

            @pl.when(i > 0)
            def _():
                pltpu.make_async_copy(eeA, dn_sh.at[dstb.at[j0]], dsA).wait()

            compute_ee(j0, eeA)
            pltpu.make_async_copy(h_hbm.at[srcb.at[j0]], rowsA, gsA).wait()

            @pl.when(i > 0)
            def _():
                pltpu.make_async_copy(rowsB, out_sh.at[dstb.at[j1]], ssB).wait()

            pltpu.async_copy(h_hbm.at[srcb.at[j1]], rowsB, gsB)
            scale_rows(rowsA, eeA)
            pltpu.async_copy(rowsA, out_sh.at[dstb.at[j0]], ssA, add=True)
            pltpu.async_copy(eeA, dn_sh.at[dstb.at[j0]], dsA, add=True)

            @pl.when(i > 0)
            def _():
                pltpu.make_async_copy(eeB, dn_sh.at[dstb.at[j1]], dsB).wait()

            compute_ee(j1, eeB)
            pltpu.make_async_copy(h_hbm.at[srcb.at[j1]], rowsB, gsB).wait()
            pltpu.make_async_copy(rowsA, out_sh.at[dstb.at[j0]], ssA).wait()
            pltpu.async_copy(h_hbm.at[srcb.at[j0 + 2]], rowsA, gsA)
            scale_rows(rowsB, eeB)
            pltpu.async_copy(rowsB, out_sh.at[dstb.at[j1]], ssB, add=True)
            pltpu.async_copy(eeB, dn_sh.at[dstb.at[j1]], dsB, add=True)

        jt = CPB - 1
        pltpu.make_async_copy(eeA, dn_sh.at[dstb.at[jt]], dsA).wait()
        compute_ee(jt, eeA)
        pltpu.make_async_copy(h_hbm.at[srcb.at[jt]], rowsA, gsA).wait()
        pltpu.make_async_copy(rowsB, out_sh.at[dstb.at[jt]], ssB).wait()
        scale_rows(rowsA, eeA)
        pltpu.sync_copy(rowsA, out_sh.at[dstb.at[jt]], add=True)
        pltpu.sync_copy(eeA, dn_sh.at[dstb.at[jt]], add=True)

    pltpu.make_async_copy(eeB, dn_sh.at[dstb.at[0]], dsB).wait()

    plsc.subcore_barrier()

    pltpu.sync_copy(out_sh.at[pl.ds(row0, RPT)],
                    part_hbm.at[core, pl.ds(row0, RPT)])
    pltpu.sync_copy(dn_sh.at[pl.ds(d0, DZ)],
                    pdn_hbm.at[core, pl.ds(d0, DZ)])

    @pl.when(sub == NS - 1)
    def _():
        pltpu.sync_copy(dn_sh.at[pl.ds(NS * DZ, N - NS * DZ)],
                        pdn_hbm.at[core, pl.ds(NS * DZ, N - NS * DZ)])


def _sc_aggregate(h, el, er, src2d, dst2d):
    mesh = plsc.VectorSubcoreMesh(core_axis_name="c", subcore_axis_name="s")
    f = functools.partial(
        pl.kernel,
        mesh=mesh,
        compiler_params=pltpu.CompilerParams(use_tc_tiling_on_sc=False,
                                             needs_layout_passes=False),
        out_type=[
            jax.ShapeDtypeStruct((NC, N, D), jnp.float32),
            jax.ShapeDtypeStruct((NC, N), jnp.float32),
        ],
        scratch_types=[
            pltpu.VMEM_SHARED((N, D), jnp.float32),
            pltpu.VMEM_SHARED((N,), jnp.float32),
            pltpu.VMEM((N,), jnp.float32),
            pltpu.VMEM((N,), jnp.float32),
            pltpu.VMEM((CPB, CH), jnp.int32),
            pltpu.VMEM((CPB, CH), jnp.int32),
            pltpu.VMEM((CH,), jnp.float32),
            pltpu.VMEM((CH,), jnp.float32),
            pltpu.VMEM((CH, D), jnp.float32),
            pltpu.VMEM((CH, D), jnp.float32),
            pltpu.SemaphoreType.DMA,
            pltpu.SemaphoreType.DMA,
            pltpu.SemaphoreType.DMA,
            pltpu.SemaphoreType.DMA,
            pltpu.SemaphoreType.DMA,
            pltpu.SemaphoreType.DMA,
        ],
    )(_sc_body)
    return f(h, el, er, src2d, dst2d)


@jax.jit
def kernel(x, edge_index, W1, al1, ar1, b1, p, W2, al2, ar2, b2, Wc, bc):
    src2d = edge_index[0].astype(jnp.int32).reshape(E // CH, CH)
    dst2d = edge_index[1].astype(jnp.int32).reshape(E // CH, CH)
    aler1 = jnp.stack([al1, ar1], axis=1)
    aler2 = jnp.stack([al2, ar2], axis=1)
    mask = jnp.clip(p, 0.0, 1.0).reshape(1, D)

    h1, eler1 = _proj1(x, W1, aler1)
    part1, pdn1 = _sc_aggregate(h1, eler1[:, 0], eler1[:, 1], src2d, dst2d)
    h2, eler2 = _proj2(part1[0], part1[1],
                       pdn1[0].reshape(N, 1), pdn1[1].reshape(N, 1),
                       b1.reshape(1, D), mask, W2, aler2)
    part2, pdn2 = _sc_aggregate(h2, eler2[:, 0], eler2[:, 1], src2d, dst2d)
    out = _final(part2[0], part2[1],
                 pdn2[0].reshape(N, 1), pdn2[1].reshape(N, 1),
                 b2.reshape(1, D), Wc, bc.reshape(1, C))
    return out.reshape(C)


# speedup vs baseline: 41.5926x; 1.0859x over previous
import functools

import jax
import jax.numpy as jnp
from jax import lax
from jax.experimental import pallas as pl
from jax.experimental.pallas import tpu as pltpu
from jax.experimental.pallas import tpu_sc as plsc

N, E, D, C = 10000, 320000, 128, 40
NC, NS = 2, 16
NW = NC * NS
EPW = E // NW
CH = 80
NCHUNK = EPW // CH
CPB = 25
NBLK = NCHUNK // CPB
PAIRS = (CPB - 1) // 2
RPT = N // NS
DZ = 624
BN = 1000


def _proj_body(x_ref, w_ref, aler_ref, h_ref, eler_ref):
    h = jnp.dot(x_ref[...], w_ref[...],
                preferred_element_type=jnp.float32,
                precision=lax.Precision.HIGHEST)
    h_ref[...] = h
    eler_ref[...] = jnp.dot(h, aler_ref[...],
                            preferred_element_type=jnp.float32,
                            precision=lax.Precision.HIGHEST)


def _proj1(x, w, aler):
    return pl.pallas_call(
        _proj_body,
        grid=(N // BN,),
        in_specs=[
            pl.BlockSpec((BN, D), lambda i: (i, 0)),
            pl.BlockSpec((D, D), lambda i: (0, 0)),
            pl.BlockSpec((D, 2), lambda i: (0, 0)),
        ],
        out_specs=[
            pl.BlockSpec((BN, D), lambda i: (i, 0)),
            pl.BlockSpec((BN, 2), lambda i: (i, 0)),
        ],
        out_shape=[
            jax.ShapeDtypeStruct((N, D), jnp.float32),
            jax.ShapeDtypeStruct((N, 2), jnp.float32),
        ],
    )(x, w, aler)


def _mid_body(pa_ref, pb_ref, dna_ref, dnb_ref, b_ref, m_ref, w_ref, aler_ref,
              h_ref, eler_ref):
    ps = pa_ref[...] + pb_ref[...]
    dn = dna_ref[...] + dnb_ref[...]
    out1 = ps / (dn + 1e-9) + b_ref[...]
    hin = jnp.where(out1 > 0, out1, jnp.exp(out1) - 1.0) * m_ref[...]
    h = jnp.dot(hin, w_ref[...],
                preferred_element_type=jnp.float32,
                precision=lax.Precision.HIGHEST)
    h_ref[...] = h
    eler_ref[...] = jnp.dot(h, aler_ref[...],
                            preferred_element_type=jnp.float32,
                            precision=lax.Precision.HIGHEST)


def _proj2(pa, pb, dna, dnb, b, mask, w, aler):
    return pl.pallas_call(
        _mid_body,
        grid=(N // BN,),
        in_specs=[
            pl.BlockSpec((BN, D), lambda i: (i, 0)),
            pl.BlockSpec((BN, D), lambda i: (i, 0)),
            pl.BlockSpec((BN, 1), lambda i: (i, 0)),
            pl.BlockSpec((BN, 1), lambda i: (i, 0)),
            pl.BlockSpec((1, D), lambda i: (0, 0)),
            pl.BlockSpec((1, D), lambda i: (0, 0)),
            pl.BlockSpec((D, D), lambda i: (0, 0)),
            pl.BlockSpec((D, 2), lambda i: (0, 0)),
        ],
        out_specs=[
            pl.BlockSpec((BN, D), lambda i: (i, 0)),
            pl.BlockSpec((BN, 2), lambda i: (i, 0)),
        ],
        out_shape=[
            jax.ShapeDtypeStruct((N, D), jnp.float32),
            jax.ShapeDtypeStruct((N, 2), jnp.float32),
        ],
    )(pa, pb, dna, dnb, b, mask, w, aler)


def _final_body(pa_ref, pb_ref, dna_ref, dnb_ref, b_ref, wc_ref, bc_ref,
                out_ref, acc_ref):
    i = pl.program_id(0)
    ps = pa_ref[...] + pb_ref[...]
    dn = dna_ref[...] + dnb_ref[...]
    out2 = ps / (dn + 1e-9) + b_ref[...]
    s = jnp.sum(out2, axis=0, keepdims=True)

    @pl.when(i == 0)
    def _():
        acc_ref[...] = jnp.zeros_like(acc_ref)

    acc_ref[...] += s

    @pl.when(i == (N // BN) - 1)
    def _():
        out_ref[...] = jnp.dot(acc_ref[...] * (1.0 / N), wc_ref[...],
                               preferred_element_type=jnp.float32,
                               precision=lax.Precision.HIGHEST) + bc_ref[...]


def _final(pa, pb, dna, dnb, b, wc, bc):
    return pl.pallas_call(
        _final_body,
        grid=(N // BN,),
        in_specs=[
            pl.BlockSpec((BN, D), lambda i: (i, 0)),
            pl.BlockSpec((BN, D), lambda i: (i, 0)),
            pl.BlockSpec((BN, 1), lambda i: (i, 0)),
            pl.BlockSpec((BN, 1), lambda i: (i, 0)),
            pl.BlockSpec((1, D), lambda i: (0, 0)),
            pl.BlockSpec((D, C), lambda i: (0, 0)),
            pl.BlockSpec((1, C), lambda i: (0, 0)),
        ],
        out_specs=pl.BlockSpec((1, C), lambda i: (0, 0)),
        out_shape=jax.ShapeDtypeStruct((1, C), jnp.float32),
        scratch_shapes=[pltpu.VMEM((1, D), jnp.float32)],
    )(pa, pb, dna, dnb, b, wc, bc)


def _sc_body(h_hbm, el_hbm, er_hbm, src_hbm, dst_hbm, part_hbm, pdn_hbm,
             out_sh, dn_sh, el_v, er_v, srcb, dstb, eeA, eeB, rowsA, rowsB,
             gsA, gsB, ssA, ssB, dsA, dsB):
    core = lax.axis_index("c")
    sub = lax.axis_index("s")
    wid = sub * NC + core

    z16 = jnp.zeros((16,), jnp.float32)

    @pl.loop(0, CH)
    def _(r):
        for j in range(D // 16):
            rowsA[r, pl.ds(j * 16, 16)] = z16

    row0 = sub * RPT
    for k in range(RPT // CH):
        pltpu.sync_copy(rowsA, out_sh.at[pl.ds(row0 + k * CH, CH)])
    pltpu.sync_copy(rowsA.at[pl.ds(0, RPT % CH)],
                    out_sh.at[pl.ds(row0 + (RPT // CH) * CH, RPT % CH)])

    for g in range(CH // 16):
        eeA[pl.ds(g * 16, 16)] = z16
    d0 = sub * DZ
    for k in range(DZ // CH):
        pltpu.sync_copy(eeA, dn_sh.at[pl.ds(d0 + k * CH, CH)])
    pltpu.sync_copy(eeA.at[pl.ds(0, DZ % CH)],
                    dn_sh.at[pl.ds(d0 + (DZ // CH) * CH, DZ % CH)])

    @pl.when(sub == NS - 1)
    def _():
        pltpu.sync_copy(eeA.at[pl.ds(0, N - NS * DZ)],
                        dn_sh.at[pl.ds(NS * DZ, N - NS * DZ)])

    plsc.subcore_barrier()

    pltpu.sync_copy(el_hbm, el_v)
    pltpu.sync_copy(er_hbm, er_v)

    def compute_ee(jrow, eebuf):
        for g in range(CH // 16):
            sv = srcb[jrow, pl.ds(g * 16, 16)]
            dv = dstb[jrow, pl.ds(g * 16, 16)]
            s = plsc.load_gather(el_v, [sv]) + plsc.load_gather(er_v, [dv])
            e = jnp.where(s > 0.0, s, 0.2 * s)
            eebuf[pl.ds(g * 16, 16)] = jnp.exp(e)

    def scale_rows(rows, eebuf):
        @plsc.parallel_loop(0, CH, unroll=4)
        def _(r):
            ridx = jnp.full((16,), 0, jnp.int32) + r
            ev = plsc.load_gather(eebuf, [ridx])
            for j in range(D // 16):
                rows[r, pl.ds(j * 16, 16)] = rows[r, pl.ds(j * 16, 16)] * ev

    @pl.loop(0, NBLK)
    def _(b):
        blk = wid * NCHUNK + b * CPB

        @pl.when(b > 0)
        def _():
            pltpu.make_async_copy(eeB, dn_sh.at[dstb.at[0]], dsB).wait()

        pltpu.sync_copy(src_hbm.at[pl.ds(blk, CPB)], srcb)
        pltpu.sync_copy(dst_hbm.at[pl.ds(blk, CPB)], dstb)

        pltpu.async_copy(h_hbm.at[srcb.at[0]], rowsA, gsA)

        @pl.loop(0, PAIRS)
        def _(i):
            j0 = 2 * i
            j1 = j0 + 1

            @pl.when(i > 0)
            def _():
                pltpu.make_async_copy(eeA, dn_sh.at[dstb.at[j0]], dsA).wait()

            compute_ee(j0, eeA)
            pltpu.make_async_copy(h_hbm.at[srcb.at[j0]], rowsA, gsA).wait()

            @pl.when(i > 0)
            def _():
                pltpu.make_async_copy(rowsB, out_sh.at[dstb.at[j1]], ssB).wait()

            pltpu.async_copy(h_hbm.at[srcb.at[j1]], rowsB, gsB)
            scale_rows(rowsA, eeA)
            pltpu.async_copy(rowsA, out_sh.at[dstb.at[j0]], ssA, add=True)
            pltpu.async_copy(eeA, dn_sh.at[dstb.at[j0]], dsA, add=True)

            @pl.when(i > 0)
            def _():
                pltpu.make_async_copy(eeB, dn_sh.at[dstb.at[j1]], dsB).wait()

            compute_ee(j1, eeB)
            pltpu.make_async_copy(h_hbm.at[srcb.at[j1]], rowsB, gsB).wait()
            pltpu.make_async_copy(rowsA, out_sh.at[dstb.at[j0]], ssA).wait()
            pltpu.async_copy(h_hbm.at[srcb.at[j0 + 2]], rowsA, gsA)
            scale_rows(rowsB, eeB)
            pltpu.async_copy(rowsB, out_sh.at[dstb.at[j1]], ssB, add=True)
            pltpu.async_copy(eeB, dn_sh.at[dstb.at[j1]], dsB, add=True)

        jt = CPB - 1
        pltpu.make_async_copy(eeA, dn_sh.at[dstb.at[jt]], dsA).wait()
        compute_ee(jt, eeA)
        pltpu.make_async_copy(h_hbm.at[srcb.at[jt]], rowsA, gsA).wait()
        pltpu.make_async_copy(rowsB, out_sh.at[dstb.at[jt]], ssB).wait()
        scale_rows(rowsA, eeA)
        pltpu.sync_copy(rowsA, out_sh.at[dstb.at[jt]], add=True)
        pltpu.sync_copy(eeA, dn_sh.at[dstb.at[jt]], add=True)

    pltpu.make_async_copy(eeB, dn_sh.at[dstb.at[0]], dsB).wait()

    plsc.subcore_barrier()

    pltpu.sync_copy(out_sh.at[pl.ds(row0, RPT)],
                    part_hbm.at[core, pl.ds(row0, RPT)])
    pltpu.sync_copy(dn_sh.at[pl.ds(d0, DZ)],
                    pdn_hbm.at[core, pl.ds(d0, DZ)])

    @pl.when(sub == NS - 1)
    def _():
        pltpu.sync_copy(dn_sh.at[pl.ds(NS * DZ, N - NS * DZ)],
                        pdn_hbm.at[core, pl.ds(NS * DZ, N - NS * DZ)])


def _sc_aggregate(h, el, er, src2d, dst2d):
    mesh = plsc.VectorSubcoreMesh(core_axis_name="c", subcore_axis_name="s")
    f = functools.partial(
        pl.kernel,
        mesh=mesh,
        compiler_params=pltpu.CompilerParams(use_tc_tiling_on_sc=False,
                                             needs_layout_passes=False),
        out_type=[
            jax.ShapeDtypeStruct((NC, N, D), jnp.float32),
            jax.ShapeDtypeStruct((NC, N), jnp.float32),
        ],
        scratch_types=[
            pltpu.VMEM_SHARED((N, D), jnp.float32),
            pltpu.VMEM_SHARED((N,), jnp.float32),
            pltpu.VMEM((N,), jnp.float32),
            pltpu.VMEM((N,), jnp.float32),
            pltpu.VMEM((CPB, CH), jnp.int32),
            pltpu.VMEM((CPB, CH), jnp.int32),
            pltpu.VMEM((CH,), jnp.float32),
            pltpu.VMEM((CH,), jnp.float32),
            pltpu.VMEM((CH, D), jnp.float32),
            pltpu.VMEM((CH, D), jnp.float32),
            pltpu.SemaphoreType.DMA,
            pltpu.SemaphoreType.DMA,
            pltpu.SemaphoreType.DMA,
            pltpu.SemaphoreType.DMA,
            pltpu.SemaphoreType.DMA,
            pltpu.SemaphoreType.DMA,
        ],
    )(_sc_body)
    return f(h, el, er, src2d, dst2d)


@jax.jit
def kernel(x, edge_index, W1, al1, ar1, b1, p, W2, al2, ar2, b2, Wc, bc):
    src2d = edge_index[0].astype(jnp.int32).reshape(E // CH, CH)
    dst2d = edge_index[1].astype(jnp.int32).reshape(E // CH, CH)
    aler1 = jnp.stack([al1, ar1], axis=1)
    aler2 = jnp.stack([al2, ar2], axis=1)
    mask = jnp.clip(p, 0.0, 1.0).reshape(1, D)

    h1, eler1 = _proj1(x, W1, aler1)
    part1, pdn1 = _sc_aggregate(h1, eler1[:, 0], eler1[:, 1], src2d, dst2d)
    h2, eler2 = _proj2(part1[0], part1[1],
                       pdn1[0].reshape(N, 1), pdn1[1].reshape(N, 1),
                       b1.reshape(1, D), mask, W2, aler2)
    part2, pdn2 = _sc_aggregate(h2, eler2[:, 0], eler2[:, 1], src2d, dst2d)
    out = _final(part2[0], part2[1],
                 pdn2[0].reshape(N, 1), pdn2[1].reshape(N, 1),
                 b2.reshape(1, D), Wc, bc.reshape(1, C))
    return out.reshape(C)


# speedup vs baseline: 41.6259x; 1.0008x over previous
import functools

import jax
import jax.numpy as jnp
from jax import lax
from jax.experimental import pallas as pl
from jax.experimental.pallas import tpu as pltpu
from jax.experimental.pallas import tpu_sc as plsc

N, E, D, C = 10000, 320000, 128, 40
NC, NS = 2, 16
NW = NC * NS
EPW = E // NW
CH = 80
NCHUNK = EPW // CH
CPB = 25
NBLK = NCHUNK // CPB
PAIRS = (CPB - 1) // 2
RPT = N // NS
DZ = 624
BN = 1000


def _proj_body(x_ref, w_ref, aler_ref, h_ref, eler_ref):
    h = jnp.dot(x_ref[...], w_ref[...],
                preferred_element_type=jnp.float32,
                precision=lax.Precision.HIGHEST)
    h_ref[...] = h
    eler_ref[...] = jnp.dot(h, aler_ref[...],
                            preferred_element_type=jnp.float32,
                            precision=lax.Precision.HIGHEST)


def _proj1(x, w, aler):
    return pl.pallas_call(
        _proj_body,
        grid=(N // BN,),
        in_specs=[
            pl.BlockSpec((BN, D), lambda i: (i, 0)),
            pl.BlockSpec((D, D), lambda i: (0, 0)),
            pl.BlockSpec((D, 2), lambda i: (0, 0)),
        ],
        out_specs=[
            pl.BlockSpec((BN, D), lambda i: (i, 0)),
            pl.BlockSpec((BN, 2), lambda i: (i, 0)),
        ],
        out_shape=[
            jax.ShapeDtypeStruct((N, D), jnp.float32),
            jax.ShapeDtypeStruct((N, 2), jnp.float32),
        ],
    )(x, w, aler)


def _mid_body(pa_ref, pb_ref, dna_ref, dnb_ref, b_ref, m_ref, w_ref, aler_ref,
              h_ref, eler_ref):
    ps = pa_ref[...] + pb_ref[...]
    dn = dna_ref[...] + dnb_ref[...]
    out1 = ps / (dn + 1e-9) + b_ref[...]
    hin = jnp.where(out1 > 0, out1, jnp.exp(out1) - 1.0) * m_ref[...]
    h = jnp.dot(hin, w_ref[...],
                preferred_element_type=jnp.float32,
                precision=lax.Precision.HIGHEST)
    h_ref[...] = h
    eler_ref[...] = jnp.dot(h, aler_ref[...],
                            preferred_element_type=jnp.float32,
                            precision=lax.Precision.HIGHEST)


def _proj2(pa, pb, dna, dnb, b, mask, w, aler):
    return pl.pallas_call(
        _mid_body,
        grid=(N // BN,),
        in_specs=[
            pl.BlockSpec((BN, D), lambda i: (i, 0)),
            pl.BlockSpec((BN, D), lambda i: (i, 0)),
            pl.BlockSpec((BN, 1), lambda i: (i, 0)),
            pl.BlockSpec((BN, 1), lambda i: (i, 0)),
            pl.BlockSpec((1, D), lambda i: (0, 0)),
            pl.BlockSpec((1, D), lambda i: (0, 0)),
            pl.BlockSpec((D, D), lambda i: (0, 0)),
            pl.BlockSpec((D, 2), lambda i: (0, 0)),
        ],
        out_specs=[
            pl.BlockSpec((BN, D), lambda i: (i, 0)),
            pl.BlockSpec((BN, 2), lambda i: (i, 0)),
        ],
        out_shape=[
            jax.ShapeDtypeStruct((N, D), jnp.float32),
            jax.ShapeDtypeStruct((N, 2), jnp.float32),
        ],
    )(pa, pb, dna, dnb, b, mask, w, aler)


def _final_body(pa_ref, pb_ref, dna_ref, dnb_ref, b_ref, wc_ref, bc_ref,
                out_ref, acc_ref):
    i = pl.program_id(0)
    ps = pa_ref[...] + pb_ref[...]
    dn = dna_ref[...] + dnb_ref[...]
    out2 = ps / (dn + 1e-9) + b_ref[...]
    s = jnp.sum(out2, axis=0, keepdims=True)

    @pl.when(i == 0)
    def _():
        acc_ref[...] = jnp.zeros_like(acc_ref)

    acc_ref[...] += s

    @pl.when(i == (N // BN) - 1)
    def _():
        out_ref[...] = jnp.dot(acc_ref[...] * (1.0 / N), wc_ref[...],
                               preferred_element_type=jnp.float32,
                               precision=lax.Precision.HIGHEST) + bc_ref[...]


def _final(pa, pb, dna, dnb, b, wc, bc):
    return pl.pallas_call(
        _final_body,
        grid=(N // BN,),
        in_specs=[
            pl.BlockSpec((BN, D), lambda i: (i, 0)),
            pl.BlockSpec((BN, D), lambda i: (i, 0)),
            pl.BlockSpec((BN, 1), lambda i: (i, 0)),
            pl.BlockSpec((BN, 1), lambda i: (i, 0)),
            pl.BlockSpec((1, D), lambda i: (0, 0)),
            pl.BlockSpec((D, C), lambda i: (0, 0)),
            pl.BlockSpec((1, C), lambda i: (0, 0)),
        ],
        out_specs=pl.BlockSpec((1, C), lambda i: (0, 0)),
        out_shape=jax.ShapeDtypeStruct((1, C), jnp.float32),
        scratch_shapes=[pltpu.VMEM((1, D), jnp.float32)],
    )(pa, pb, dna, dnb, b, wc, bc)


def _sc_body(h_hbm, el_hbm, er_hbm, src_hbm, dst_hbm, part_hbm, pdn_hbm,
             out_sh, dn_sh, el_v, er_v, srcb, dstb, eeA, eeB, rowsA, rowsB,
             gsA, gsB, ssA, ssB, dsA, dsB):
    core = lax.axis_index("c")
    sub = lax.axis_index("s")
    wid = sub * NC + core

    z16 = jnp.zeros((16,), jnp.float32)

    @pl.loop(0, CH)
    def _(r):
        for j in range(D // 16):
            rowsA[r, pl.ds(j * 16, 16)] = z16

    row0 = sub * RPT
    for k in range(RPT // CH):
        pltpu.sync_copy(rowsA, out_sh.at[pl.ds(row0 + k * CH, CH)])
    pltpu.sync_copy(rowsA.at[pl.ds(0, RPT % CH)],
                    out_sh.at[pl.ds(row0 + (RPT // CH) * CH, RPT % CH)])

    for g in range(CH // 16):
        eeA[pl.ds(g * 16, 16)] = z16
    d0 = sub * DZ
    for k in range(DZ // CH):
        pltpu.sync_copy(eeA, dn_sh.at[pl.ds(d0 + k * CH, CH)])
    pltpu.sync_copy(eeA.at[pl.ds(0, DZ % CH)],
                    dn_sh.at[pl.ds(d0 + (DZ // CH) * CH, DZ % CH)])

    @pl.when(sub == NS - 1)
    def _():
        pltpu.sync_copy(eeA.at[pl.ds(0, N - NS * DZ)],
                        dn_sh.at[pl.ds(NS * DZ, N - NS * DZ)])

    plsc.subcore_barrier()

    pltpu.sync_copy(el_hbm, el_v)
    pltpu.sync_copy(er_hbm, er_v)

    def compute_ee(jrow, eebuf):
        for g in range(CH // 16):
            sv = srcb[jrow, pl.ds(g * 16, 16)]
            dv = dstb[jrow, pl.ds(g * 16, 16)]
            s = plsc.load_gather(el_v, [sv]) + plsc.load_gather(er_v, [dv])
            e = jnp.where(s > 0.0, s, 0.2 * s)
            eebuf[pl.ds(g * 16, 16)] = jnp.exp(e)

    def scale_rows(rows, eebuf):
        @plsc.parallel_loop(0, CH, unroll=8)
        def _(r):
            ridx = jnp.full((16,), 0, jnp.int32) + r
            ev = plsc.load_gather(eebuf, [ridx])
            for j in range(D // 16):
                rows[r, pl.ds(j * 16, 16)] = rows[r, pl.ds(j * 16, 16)] * ev

    @pl.loop(0, NBLK)
    def _(b):
        blk = wid * NCHUNK + b * CPB

        @pl.when(b > 0)
        def _():
            pltpu.make_async_copy(eeB, dn_sh.at[dstb.at[0]], dsB).wait()

        pltpu.sync_copy(src_hbm.at[pl.ds(blk, CPB)], srcb)
        pltpu.sync_copy(dst_hbm.at[pl.ds(blk, CPB)], dstb)

        pltpu.async_copy(h_hbm.at[srcb.at[0]], rowsA, gsA)

        @pl.loop(0, PAIRS)
        def _(i):
            j0 = 2 * i
            j1 = j0 + 1

            @pl.when(i > 0)
            def _():
                pltpu.make_async_copy(eeA, dn_sh.at[dstb.at[j0]], dsA).wait()

            compute_ee(j0, eeA)
            pltpu.make_async_copy(h_hbm.at[srcb.at[j0]], rowsA, gsA).wait()

            @pl.when(i > 0)
            def _():
                pltpu.make_async_copy(rowsB, out_sh.at[dstb.at[j1]], ssB).wait()

            pltpu.async_copy(h_hbm.at[srcb.at[j1]], rowsB, gsB)
            scale_rows(rowsA, eeA)
            pltpu.async_copy(rowsA, out_sh.at[dstb.at[j0]], ssA, add=True)
            pltpu.async_copy(eeA, dn_sh.at[dstb.at[j0]], dsA, add=True)

            @pl.when(i > 0)
            def _():
                pltpu.make_async_copy(eeB, dn_sh.at[dstb.at[j1]], dsB).wait()

            compute_ee(j1, eeB)
            pltpu.make_async_copy(h_hbm.at[srcb.at[j1]], rowsB, gsB).wait()
            pltpu.make_async_copy(rowsA, out_sh.at[dstb.at[j0]], ssA).wait()
            pltpu.async_copy(h_hbm.at[srcb.at[j0 + 2]], rowsA, gsA)
            scale_rows(rowsB, eeB)
            pltpu.async_copy(rowsB, out_sh.at[dstb.at[j1]], ssB, add=True)
            pltpu.async_copy(eeB, dn_sh.at[dstb.at[j1]], dsB, add=True)

        jt = CPB - 1
        pltpu.make_async_copy(eeA, dn_sh.at[dstb.at[jt]], dsA).wait()
        compute_ee(jt, eeA)
        pltpu.make_async_copy(h_hbm.at[srcb.at[jt]], rowsA, gsA).wait()
        pltpu.make_async_copy(rowsB, out_sh.at[dstb.at[jt]], ssB).wait()
        scale_rows(rowsA, eeA)
        pltpu.sync_copy(rowsA, out_sh.at[dstb.at[jt]], add=True)
        pltpu.sync_copy(eeA, dn_sh.at[dstb.at[jt]], add=True)

    pltpu.make_async_copy(eeB, dn_sh.at[dstb.at[0]], dsB).wait()

    plsc.subcore_barrier()

    pltpu.sync_copy(out_sh.at[pl.ds(row0, RPT)],
                    part_hbm.at[core, pl.ds(row0, RPT)])
    pltpu.sync_copy(dn_sh.at[pl.ds(d0, DZ)],
                    pdn_hbm.at[core, pl.ds(d0, DZ)])

    @pl.when(sub == NS - 1)
    def _():
        pltpu.sync_copy(dn_sh.at[pl.ds(NS * DZ, N - NS * DZ)],
                        pdn_hbm.at[core, pl.ds(NS * DZ, N - NS * DZ)])


def _sc_aggregate(h, el, er, src2d, dst2d):
    mesh = plsc.VectorSubcoreMesh(core_axis_name="c", subcore_axis_name="s")
    f = functools.partial(
        pl.kernel,
        mesh=mesh,
        compiler_params=pltpu.CompilerParams(use_tc_tiling_on_sc=False,
                                             needs_layout_passes=False),
        out_type=[
            jax.ShapeDtypeStruct((NC, N, D), jnp.float32),
            jax.ShapeDtypeStruct((NC, N), jnp.float32),
        ],
        scratch_types=[
            pltpu.VMEM_SHARED((N, D), jnp.float32),
            pltpu.VMEM_SHARED((N,), jnp.float32),
            pltpu.VMEM((N,), jnp.float32),
            pltpu.VMEM((N,), jnp.float32),
            pltpu.VMEM((CPB, CH), jnp.int32),
            pltpu.VMEM((CPB, CH), jnp.int32),
            pltpu.VMEM((CH,), jnp.float32),
            pltpu.VMEM((CH,), jnp.float32),
            pltpu.VMEM((CH, D), jnp.float32),
            pltpu.VMEM((CH, D), jnp.float32),
            pltpu.SemaphoreType.DMA,
            pltpu.SemaphoreType.DMA,
            pltpu.SemaphoreType.DMA,
            pltpu.SemaphoreType.DMA,
            pltpu.SemaphoreType.DMA,
            pltpu.SemaphoreType.DMA,
        ],
    )(_sc_body)
    return f(h, el, er, src2d, dst2d)


@jax.jit
def kernel(x, edge_index, W1, al1, ar1, b1, p, W2, al2, ar2, b2, Wc, bc):
    src2d = edge_index[0].astype(jnp.int32).reshape(E // CH, CH)
    dst2d = edge_index[1].astype(jnp.int32).reshape(E // CH, CH)
    aler1 = jnp.stack([al1, ar1], axis=1)
    aler2 = jnp.stack([al2, ar2], axis=1)
    mask = jnp.clip(p, 0.0, 1.0).reshape(1, D)

    h1, eler1 = _proj1(x, W1, aler1)
    part1, pdn1 = _sc_aggregate(h1, eler1[:, 0], eler1[:, 1], src2d, dst2d)
    h2, eler2 = _proj2(part1[0], part1[1],
                       pdn1[0].reshape(N, 1), pdn1[1].reshape(N, 1),
                       b1.reshape(1, D), mask, W2, aler2)
    part2, pdn2 = _sc_aggregate(h2, eler2[:, 0], eler2[:, 1], src2d, dst2d)
    out = _final(part2[0], part2[1],
                 pdn2[0].reshape(N, 1), pdn2[1].reshape(N, 1),
                 b2.reshape(1, D), Wc, bc.reshape(1, C))
    return out.reshape(C)


# speedup vs baseline: 41.6292x; 1.0001x over previous
import functools

import jax
import jax.numpy as jnp
from jax import lax
from jax.experimental import pallas as pl
from jax.experimental.pallas import tpu as pltpu
from jax.experimental.pallas import tpu_sc as plsc

N, E, D, C = 10000, 320000, 128, 40
NC, NS = 2, 16
NW = NC * NS
EPW = E // NW
CH = 80
NCHUNK = EPW // CH
CPB = 25
NBLK = NCHUNK // CPB
PAIRS = (CPB - 1) // 2
RPT = N // NS
DZ = 624
BN = 1000


def _proj_body(x_ref, w_ref, aler_ref, h_ref, eler_ref):
    h = jnp.dot(x_ref[...], w_ref[...],
                preferred_element_type=jnp.float32,
                precision=lax.Precision.HIGHEST)
    h_ref[...] = h
    eler_ref[...] = jnp.dot(h, aler_ref[...],
                            preferred_element_type=jnp.float32,
                            precision=lax.Precision.HIGHEST)


def _proj1(x, w, aler):
    return pl.pallas_call(
        _proj_body,
        grid=(N // BN,),
        in_specs=[
            pl.BlockSpec((BN, D), lambda i: (i, 0)),
            pl.BlockSpec((D, D), lambda i: (0, 0)),
            pl.BlockSpec((D, 2), lambda i: (0, 0)),
        ],
        out_specs=[
            pl.BlockSpec((BN, D), lambda i: (i, 0)),
            pl.BlockSpec((BN, 2), lambda i: (i, 0)),
        ],
        out_shape=[
            jax.ShapeDtypeStruct((N, D), jnp.float32),
            jax.ShapeDtypeStruct((N, 2), jnp.float32),
        ],
    )(x, w, aler)


def _mid_body(pa_ref, pb_ref, dna_ref, dnb_ref, b_ref, m_ref, w_ref, aler_ref,
              h_ref, eler_ref):
    ps = pa_ref[...] + pb_ref[...]
    dn = dna_ref[...] + dnb_ref[...]
    out1 = ps / (dn + 1e-9) + b_ref[...]
    hin = jnp.where(out1 > 0, out1, jnp.exp(out1) - 1.0) * m_ref[...]
    h = jnp.dot(hin, w_ref[...],
                preferred_element_type=jnp.float32,
                precision=lax.Precision.HIGHEST)
    h_ref[...] = h
    eler_ref[...] = jnp.dot(h, aler_ref[...],
                            preferred_element_type=jnp.float32,
                            precision=lax.Precision.HIGHEST)


def _proj2(pa, pb, dna, dnb, b, mask, w, aler):
    return pl.pallas_call(
        _mid_body,
        grid=(N // BN,),
        in_specs=[
            pl.BlockSpec((BN, D), lambda i: (i, 0)),
            pl.BlockSpec((BN, D), lambda i: (i, 0)),
            pl.BlockSpec((BN, 1), lambda i: (i, 0)),
            pl.BlockSpec((BN, 1), lambda i: (i, 0)),
            pl.BlockSpec((1, D), lambda i: (0, 0)),
            pl.BlockSpec((1, D), lambda i: (0, 0)),
            pl.BlockSpec((D, D), lambda i: (0, 0)),
            pl.BlockSpec((D, 2), lambda i: (0, 0)),
        ],
        out_specs=[
            pl.BlockSpec((BN, D), lambda i: (i, 0)),
            pl.BlockSpec((BN, 2), lambda i: (i, 0)),
        ],
        out_shape=[
            jax.ShapeDtypeStruct((N, D), jnp.float32),
            jax.ShapeDtypeStruct((N, 2), jnp.float32),
        ],
    )(pa, pb, dna, dnb, b, mask, w, aler)


def _final_body(pa_ref, pb_ref, dna_ref, dnb_ref, b_ref, wc_ref, bc_ref,
                out_ref, acc_ref):
    i = pl.program_id(0)
    ps = pa_ref[...] + pb_ref[...]
    dn = dna_ref[...] + dnb_ref[...]
    out2 = ps / (dn + 1e-9) + b_ref[...]
    s = jnp.sum(out2, axis=0, keepdims=True)

    @pl.when(i == 0)
    def _():
        acc_ref[...] = jnp.zeros_like(acc_ref)

    acc_ref[...] += s

    @pl.when(i == (N // BN) - 1)
    def _():
        out_ref[...] = jnp.dot(acc_ref[...] * (1.0 / N), wc_ref[...],
                               preferred_element_type=jnp.float32,
                               precision=lax.Precision.HIGHEST) + bc_ref[...]


def _final(pa, pb, dna, dnb, b, wc, bc):
    return pl.pallas_call(
        _final_body,
        grid=(N // BN,),
        in_specs=[
            pl.BlockSpec((BN, D), lambda i: (i, 0)),
            pl.BlockSpec((BN, D), lambda i: (i, 0)),
            pl.BlockSpec((BN, 1), lambda i: (i, 0)),
            pl.BlockSpec((BN, 1), lambda i: (i, 0)),
            pl.BlockSpec((1, D), lambda i: (0, 0)),
            pl.BlockSpec((D, C), lambda i: (0, 0)),
            pl.BlockSpec((1, C), lambda i: (0, 0)),
        ],
        out_specs=pl.BlockSpec((1, C), lambda i: (0, 0)),
        out_shape=jax.ShapeDtypeStruct((1, C), jnp.float32),
        scratch_shapes=[pltpu.VMEM((1, D), jnp.float32)],
    )(pa, pb, dna, dnb, b, wc, bc)


def _sc_body(h_hbm, el_hbm, er_hbm, src_hbm, dst_hbm, part_hbm, pdn_hbm,
             out_sh, dn_sh, el_v, er_v, srcb, dstb, eeA, eeB, rowsA, rowsB,
             gsA, gsB, ssA, ssB, dsA, dsB):
    core = lax.axis_index("c")
    sub = lax.axis_index("s")
    wid = sub * NC + core

    z16 = jnp.zeros((16,), jnp.float32)

    @pl.loop(0, CH)
    def _(r):
        for j in range(D // 16):
            rowsA[r, pl.ds(j * 16, 16)] = z16

    row0 = sub * RPT
    for k in range(RPT // CH):
        pltpu.sync_copy(rowsA, out_sh.at[pl.ds(row0 + k * CH, CH)])
    pltpu.sync_copy(rowsA.at[pl.ds(0, RPT % CH)],
                    out_sh.at[pl.ds(row0 + (RPT // CH) * CH, RPT % CH)])

    for g in range(CH // 16):
        eeA[pl.ds(g * 16, 16)] = z16
    d0 = sub * DZ
    for k in range(DZ // CH):
        pltpu.sync_copy(eeA, dn_sh.at[pl.ds(d0 + k * CH, CH)])
    pltpu.sync_copy(eeA.at[pl.ds(0, DZ % CH)],
                    dn_sh.at[pl.ds(d0 + (DZ // CH) * CH, DZ % CH)])

    @pl.when(sub == NS - 1)
    def _():
        pltpu.sync_copy(eeA.at[pl.ds(0, N - NS * DZ)],
                        dn_sh.at[pl.ds(NS * DZ, N - NS * DZ)])

    plsc.subcore_barrier()

    pltpu.sync_copy(el_hbm, el_v)
    pltpu.sync_copy(er_hbm, er_v)

    def compute_ee(jrow, eebuf):
        for g in range(CH // 16):
            sv = srcb[jrow, pl.ds(g * 16, 16)]
            dv = dstb[jrow, pl.ds(g * 16, 16)]
            s = plsc.load_gather(el_v, [sv]) + plsc.load_gather(er_v, [dv])
            e = jnp.where(s > 0.0, s, 0.2 * s)
            eebuf[pl.ds(g * 16, 16)] = jnp.exp(e)

    def scale_rows(rows, eebuf):
        @plsc.parallel_loop(0, CH, unroll=4)
        def _(r):
            ridx = jnp.full((16,), 0, jnp.int32) + r
            ev = plsc.load_gather(eebuf, [ridx])
            for j in range(D // 16):
                rows[r, pl.ds(j * 16, 16)] = rows[r, pl.ds(j * 16, 16)] * ev

    @pl.loop(0, NBLK)
    def _(b):
        blk = wid * NCHUNK + b * CPB

        @pl.when(b > 0)
        def _():
            pltpu.make_async_copy(eeB, dn_sh.at[dstb.at[0]], dsB).wait()

        pltpu.sync_copy(src_hbm.at[pl.ds(blk, CPB)], srcb)
        pltpu.sync_copy(dst_hbm.at[pl.ds(blk, CPB)], dstb)

        pltpu.async_copy(h_hbm.at[srcb.at[0]], rowsA, gsA)

        @pl.loop(0, PAIRS)
        def _(i):
            j0 = 2 * i
            j1 = j0 + 1

            @pl.when(i > 0)
            def _():
                pltpu.make_async_copy(eeA, dn_sh.at[dstb.at[j0]], dsA).wait()

            compute_ee(j0, eeA)
            pltpu.make_async_copy(h_hbm.at[srcb.at[j0]], rowsA, gsA).wait()

            @pl.when(i > 0)
            def _():
                pltpu.make_async_copy(rowsB, out_sh.at[dstb.at[j1]], ssB).wait()

            pltpu.async_copy(h_hbm.at[srcb.at[j1]], rowsB, gsB)
            scale_rows(rowsA, eeA)
            pltpu.async_copy(rowsA, out_sh.at[dstb.at[j0]], ssA, add=True)
            pltpu.async_copy(eeA, dn_sh.at[dstb.at[j0]], dsA, add=True)

            @pl.when(i > 0)
            def _():
                pltpu.make_async_copy(eeB, dn_sh.at[dstb.at[j1]], dsB).wait()

            compute_ee(j1, eeB)
            pltpu.make_async_copy(h_hbm.at[srcb.at[j1]], rowsB, gsB).wait()
            pltpu.make_async_copy(rowsA, out_sh.at[dstb.at[j0]], ssA).wait()
            pltpu.async_copy(h_hbm.at[srcb.at[j0 + 2]], rowsA, gsA)
            scale_rows(rowsB, eeB)
            pltpu.async_copy(rowsB, out_sh.at[dstb.at[j1]], ssB, add=True)
            pltpu.async_copy(eeB, dn_sh.at[dstb.at[j1]], dsB, add=True)

        jt = CPB - 1
        pltpu.make_async_copy(eeA, dn_sh.at[dstb.at[jt]], dsA).wait()
        compute_ee(jt, eeA)
        pltpu.make_async_copy(h_hbm.at[srcb.at[jt]], rowsA, gsA).wait()
        pltpu.make_async_copy(rowsB, out_sh.at[dstb.at[jt]], ssB).wait()
        scale_rows(rowsA, eeA)
        pltpu.sync_copy(rowsA, out_sh.at[dstb.at[jt]], add=True)
        pltpu.sync_copy(eeA, dn_sh.at[dstb.at[jt]], add=True)

    pltpu.make_async_copy(eeB, dn_sh.at[dstb.at[0]], dsB).wait()

    plsc.subcore_barrier()

    pltpu.sync_copy(out_sh.at[pl.ds(row0, RPT)],
                    part_hbm.at[core, pl.ds(row0, RPT)])
    pltpu.sync_copy(dn_sh.at[pl.ds(d0, DZ)],
                    pdn_hbm.at[core, pl.ds(d0, DZ)])

    @pl.when(sub == NS - 1)
    def _():
        pltpu.sync_copy(dn_sh.at[pl.ds(NS * DZ, N - NS * DZ)],
                        pdn_hbm.at[core, pl.ds(NS * DZ, N - NS * DZ)])


def _sc_aggregate(h, el, er, src2d, dst2d):
    mesh = plsc.VectorSubcoreMesh(core_axis_name="c", subcore_axis_name="s")
    f = functools.partial(
        pl.kernel,
        mesh=mesh,
        compiler_params=pltpu.CompilerParams(use_tc_tiling_on_sc=False,
                                             needs_layout_passes=False),
        out_type=[
            jax.ShapeDtypeStruct((NC, N, D), jnp.float32),
            jax.ShapeDtypeStruct((NC, N), jnp.float32),
        ],
        scratch_types=[
            pltpu.VMEM_SHARED((N, D), jnp.float32),
            pltpu.VMEM_SHARED((N,), jnp.float32),
            pltpu.VMEM((N,), jnp.float32),
            pltpu.VMEM((N,), jnp.float32),
            pltpu.VMEM((CPB, CH), jnp.int32),
            pltpu.VMEM((CPB, CH), jnp.int32),
            pltpu.VMEM((CH,), jnp.float32),
            pltpu.VMEM((CH,), jnp.float32),
            pltpu.VMEM((CH, D), jnp.float32),
            pltpu.VMEM((CH, D), jnp.float32),
            pltpu.SemaphoreType.DMA,
            pltpu.SemaphoreType.DMA,
            pltpu.SemaphoreType.DMA,
            pltpu.SemaphoreType.DMA,
            pltpu.SemaphoreType.DMA,
            pltpu.SemaphoreType.DMA,
        ],
    )(_sc_body)
    return f(h, el, er, src2d, dst2d)


@jax.jit
def kernel(x, edge_index, W1, al1, ar1, b1, p, W2, al2, ar2, b2, Wc, bc):
    src2d = edge_index[0].astype(jnp.int32).reshape(E // CH, CH)
    dst2d = edge_index[1].astype(jnp.int32).reshape(E // CH, CH)
    aler1 = jnp.stack([al1, ar1], axis=1)
    aler2 = jnp.stack([al2, ar2], axis=1)
    mask = jnp.clip(p, 0.0, 1.0).reshape(1, D)

    h1, eler1 = _proj1(x, W1, aler1)
    part1, pdn1 = _sc_aggregate(h1, eler1[:, 0], eler1[:, 1], src2d, dst2d)
    h2, eler2 = _proj2(part1[0], part1[1],
                       pdn1[0].reshape(N, 1), pdn1[1].reshape(N, 1),
                       b1.reshape(1, D), mask, W2, aler2)
    part2, pdn2 = _sc_aggregate(h2, eler2[:, 0], eler2[:, 1], src2d, dst2d)
    out = _final(part2[0], part2[1],
                 pdn2[0].reshape(N, 1), pdn2[1].reshape(N, 1),
                 b2.reshape(1, D), Wc, bc.reshape(1, C))
    return out.reshape(C)
